# baseline (speedup 1.0000x reference)
import functools

import jax
import jax.numpy as jnp
from jax import lax
from jax.experimental import pallas as pl
from jax.experimental.pallas import tpu as pltpu
from jax.experimental.pallas import tpu_sc as plsc

_N = 10000
_E = 320000
_NP = 10240
_NC = 2
_NS = 16
_NW = _NC * _NS
_EW = _E // _NW
_CHUNK = 80
_NCHUNK = _EW // _CHUNK
_RSUB = _NP // _NS
_ZROWS = 32
_ZN = _RSUB // _ZROWS
_WROWS = 128
_WB = _RSUB // _WROWS


def _sc_mesh():
    return plsc.VectorSubcoreMesh(core_axis_name="c", subcore_axis_name="s")


def _make_agg(D):
    @functools.partial(
        pl.kernel,
        out_type=jax.ShapeDtypeStruct((_NC, _NP, D), jnp.float32),
        mesh=_sc_mesh(),
        scratch_types=[
            pltpu.VMEM((_CHUNK,), jnp.int32),
            pltpu.VMEM((_CHUNK,), jnp.int32),
            pltpu.VMEM((_CHUNK,), jnp.int32),
            pltpu.VMEM((_CHUNK,), jnp.int32),
            pltpu.VMEM((_CHUNK, D), jnp.float32),
            pltpu.VMEM((_CHUNK, D), jnp.float32),
            pltpu.VMEM((_ZROWS, D), jnp.float32),
            pltpu.VMEM_SHARED((_NP, D), jnp.float32),
            pltpu.SemaphoreType.DMA,
            pltpu.SemaphoreType.DMA,
            pltpu.SemaphoreType.DMA,
            pltpu.SemaphoreType.DMA,
        ],
    )
    def agg(hp_hbm, src_hbm, dst_hbm, out_hbm, sa, da, sb, db, ma, mb, zv,
            acc, semia, semib, sema, semb):
        c = lax.axis_index("c")
        s = lax.axis_index("s")
        base = (c * _NS + s) * _EW

        def ldx(i, sv, dv, sem):
            off = base + i * _CHUNK
            ca = pltpu.make_async_copy(src_hbm.at[pl.ds(off, _CHUNK)], sv, sem)
            cb = pltpu.make_async_copy(dst_hbm.at[pl.ds(off, _CHUNK)], dv, sem)
            return ca, cb

        def ldx_start(i, sv, dv, sem):
            ca, cb = ldx(i, sv, dv, sem)
            ca.start()
            cb.start()

        def ldx_wait(i, sv, dv, sem):
            ca, cb = ldx(i, sv, dv, sem)
            ca.wait()
            cb.wait()

        def gather(i, sv, buf, sem):
            return pltpu.make_async_copy(hp_hbm.at[sv], buf, sem)

        ldx_start(0, sa, da, semia)

        @pl.loop(0, _ZROWS)
        def _zero(i):
            for j in range(D // 16):
                zv[i, pl.ds(j * 16, 16)] = jnp.zeros((16,), jnp.float32)

        @pl.loop(0, _ZN)
        def _clear(k):
            pltpu.sync_copy(zv, acc.at[pl.ds(s * _RSUB + k * _ZROWS, _ZROWS)])

        plsc.subcore_barrier()

        ldx_wait(0, sa, da, semia)
        gather(0, sa, ma, sema).start()
        ldx_start(1, sb, db, semib)

        @pl.loop(0, (_NCHUNK - 3) // 2)
        def _edges(k):
            i = k * 2
            ldx_wait(i + 1, sb, db, semib)
            gather(i + 1, sb, mb, semb).start()
            gather(i, sa, ma, sema).wait()
            pltpu.sync_copy(ma, acc.at[da], add=True)
            ldx_start(i + 2, sa, da, semia)
            ldx_wait(i + 2, sa, da, semia)
            gather(i + 2, sa, ma, sema).start()
            gather(i + 1, sb, mb, semb).wait()
            pltpu.sync_copy(mb, acc.at[db], add=True)
            ldx_start(i + 3, sb, db, semib)

        i0 = _NCHUNK - 3
        ldx_wait(i0 + 1, sb, db, semib)
        gather(i0 + 1, sb, mb, semb).start()
        gather(i0, sa, ma, sema).wait()
        pltpu.sync_copy(ma, acc.at[da], add=True)
        ldx_start(i0 + 2, sa, da, semia)
        ldx_wait(i0 + 2, sa, da, semia)
        gather(i0 + 2, sa, ma, sema).start()
        gather(i0 + 1, sb, mb, semb).wait()
        pltpu.sync_copy(mb, acc.at[db], add=True)
        gather(i0 + 2, sa, ma, sema).wait()
        pltpu.sync_copy(ma, acc.at[da], add=True)

        plsc.subcore_barrier()

        @pl.loop(0, _WB)
        def _writeback(k):
            r0 = s * _RSUB + k * _WROWS
            pltpu.sync_copy(acc.at[pl.ds(r0, _WROWS)],
                            out_hbm.at[c, pl.ds(r0, _WROWS)])

    return agg


def _make_deg():
    D = 16

    @functools.partial(
        pl.kernel,
        out_type=jax.ShapeDtypeStruct((_NC, _NP, D), jnp.float32),
        mesh=_sc_mesh(),
        scratch_types=[
            pltpu.VMEM((_CHUNK,), jnp.int32),
            pltpu.VMEM((_CHUNK,), jnp.int32),
            pltpu.VMEM((_CHUNK, D), jnp.float32),
            pltpu.VMEM((_ZROWS, D), jnp.float32),
            pltpu.VMEM_SHARED((_NP, D), jnp.float32),
            pltpu.SemaphoreType.DMA,
            pltpu.SemaphoreType.DMA,
            pltpu.SemaphoreType.DMA,
            pltpu.SemaphoreType.DMA,
        ],
    )
    def deg(dst_hbm, out_hbm, da, db, onesv, zv, acc, semia, semib, sema,
            semb):
        c = lax.axis_index("c")
        s = lax.axis_index("s")
        base = (c * _NS + s) * _EW

        def ldx(i, dv, sem):
            off = base + i * _CHUNK
            return pltpu.make_async_copy(dst_hbm.at[pl.ds(off, _CHUNK)], dv,
                                         sem)

        def scat(dv, sem):
            return pltpu.make_async_copy(onesv, acc.at[dv], sem)

        ldx(0, da, semia).start()

        @pl.loop(0, _ZROWS)
        def _zero(i):
            zv[i, pl.ds(0, 16)] = jnp.zeros((16,), jnp.float32)

        @pl.loop(0, _CHUNK)
        def _fill(i):
            onesv[i, pl.ds(0, 16)] = jnp.full((16,), 1.0, jnp.float32)

        @pl.loop(0, _ZN)
        def _clear(k):
            pltpu.sync_copy(zv, acc.at[pl.ds(s * _RSUB + k * _ZROWS, _ZROWS)])

        plsc.subcore_barrier()

        ldx(0, da, semia).wait()
        scat(da, sema).start(add=True)
        ldx(1, db, semib).start()

        @pl.loop(0, (_NCHUNK - 3) // 2)
        def _edges(k):
            i = k * 2
            ldx(i + 1, db, semib).wait()
            scat(db, semb).start(add=True)
            scat(da, sema).wait()
            ldx(i + 2, da, semia).start()
            ldx(i + 2, da, semia).wait()
            scat(da, sema).start(add=True)
            scat(db, semb).wait()
            ldx(i + 3, db, semib).start()

        i0 = _NCHUNK - 3
        ldx(i0 + 1, db, semib).wait()
        scat(db, semb).start(add=True)
        scat(da, sema).wait()
        ldx(i0 + 2, da, semia).start()
        ldx(i0 + 2, da, semia).wait()
        scat(da, sema).start(add=True)
        scat(db, semb).wait()
        scat(da, sema).wait()

        plsc.subcore_barrier()

        @pl.loop(0, _WB)
        def _writeback(k):
            r0 = s * _RSUB + k * _WROWS
            pltpu.sync_copy(acc.at[pl.ds(r0, _WROWS)],
                            out_hbm.at[c, pl.ds(r0, _WROWS)])

    return deg


_BR = 512
_GRID = _NP // _BR


def _dinv(degp_blk):
    deg = degp_blk[0, :, 0] + degp_blk[1, :, 0] + 1.0
    return lax.rsqrt(deg)


def _mm(a, b):
    return jnp.dot(a, b, preferred_element_type=jnp.float32)


def _first_layer_pre(degp_ref, x_ref, w_ref, hp_ref):
    di = _dinv(degp_ref[...])
    hp_ref[...] = _mm(x_ref[...], w_ref[...]) * di[:, None]


def _mid_layer(degp_ref, agg_ref, hp_ref, b_ref, w_ref, out_ref):
    di = _dinv(degp_ref[...])
    a = agg_ref[0] + agg_ref[1] + hp_ref[...]
    x = jnp.tanh(a * di[:, None] + b_ref[...])
    out_ref[...] = _mm(x, w_ref[...]) * di[:, None]


def _final_layer(degp_ref, agg_ref, hp_ref, b_ref, wc_ref, bc_ref, out_ref):
    di = _dinv(degp_ref[...])
    a = agg_ref[0] + agg_ref[1] + hp_ref[...]
    x = jnp.tanh(a * di[:, None] + b_ref[...])
    out_ref[...] = _mm(x, wc_ref[...]) + bc_ref[...]


def _row_spec(D):
    return pl.BlockSpec((_BR, D), lambda i: (i, 0))


def _agg_spec(D):
    return pl.BlockSpec((_NC, _BR, D), lambda i: (0, i, 0))


def _full_spec(shape):
    return pl.BlockSpec(shape, lambda i: tuple(0 for _ in shape))


def _tc_pre(degp, x, w):
    return pl.pallas_call(
        _first_layer_pre,
        grid=(_GRID,),
        in_specs=[_agg_spec(16), _row_spec(128), _full_spec((128, 128))],
        out_specs=_row_spec(128),
        out_shape=jax.ShapeDtypeStruct((_NP, 128), jnp.float32),
    )(degp, x, w)


def _tc_mid(degp, aggp, hp, b, w, dout):
    din = hp.shape[1]
    return pl.pallas_call(
        _mid_layer,
        grid=(_GRID,),
        in_specs=[_agg_spec(16), _agg_spec(din), _row_spec(din),
                  _full_spec((1, din)), _full_spec((din, dout))],
        out_specs=_row_spec(dout),
        out_shape=jax.ShapeDtypeStruct((_NP, dout), jnp.float32),
    )(degp, aggp, hp, b, w)


def _tc_final(degp, aggp, hp, b, wc, bc):
    return pl.pallas_call(
        _final_layer,
        grid=(_GRID,),
        in_specs=[_agg_spec(16), _agg_spec(128), _row_spec(128),
                  _full_spec((1, 128)), _full_spec((128, 128)),
                  _full_spec((1, 128))],
        out_specs=_row_spec(128),
        out_shape=jax.ShapeDtypeStruct((_NP, 128), jnp.float32),
    )(degp, aggp, hp, b, wc, bc)


_deg_kernel = _make_deg()
_agg128 = _make_agg(128)


def kernel(x, edge_index, W1, b1, W2, b2, W3, b3, Wc, bc):
    src = edge_index[0]
    dst = edge_index[1]
    xp = jnp.pad(x, ((0, _NP - _N), (0, 0)))

    degp = _deg_kernel(dst)
    hp1 = _tc_pre(degp, xp, W1)
    a1 = _agg128(hp1, src, dst)
    hp2 = _tc_mid(degp, a1, hp1, b1.reshape(1, -1), W2, 128)
    a2 = _agg128(hp2, src, dst)
    w3p = jnp.pad(W3, ((0, 0), (0, 128 - W3.shape[1])))
    b3p = jnp.pad(b3, (0, 128 - b3.shape[0]))
    hp3 = _tc_mid(degp, a2, hp2, b2.reshape(1, -1), w3p, 128)
    a3 = _agg128(hp3, src, dst)

    wcp = jnp.pad(Wc, ((0, 128 - Wc.shape[0]), (0, 128 - Wc.shape[1])))
    bcp = jnp.pad(bc, (0, 128 - bc.shape[0])).reshape(1, -1)
    out = _tc_final(degp, a3, hp3, b3p.reshape(1, -1), wcp, bcp)
    return out[:_N, :Wc.shape[1]]
